# baseline (speedup 1.0000x reference)
import functools

import jax
import jax.numpy as jnp
from jax import lax
from jax.experimental import pallas as pl
from jax.experimental.pallas import tpu as pltpu
from jax.experimental.pallas import tpu_sc as plsc

EMB_DIM = 16
SC_ROWS = 8
CHUNK_COLS = 3200
TC_BLOCK_COLS = 32000


@functools.cache
def _build_sc_half(n_edges: int, emb_dim: int):
    info = plsc.get_sparse_core_info()
    num_workers = info.num_cores * info.num_subcores
    lanes = info.num_lanes
    assert n_edges % CHUNK_COLS == 0
    n_chunks = n_edges // CHUNK_COLS

    mesh = plsc.VectorSubcoreMesh(core_axis_name="c", subcore_axis_name="s")

    @functools.partial(
        pl.kernel,
        mesh=mesh,
        out_type=jax.ShapeDtypeStruct((SC_ROWS, n_edges), jnp.float32),
        scratch_types=[
            pltpu.VMEM((2 * emb_dim,), jnp.float32),
            pltpu.VMEM((SC_ROWS, CHUNK_COLS), jnp.float32),
            pltpu.SemaphoreType.DMA,
        ],
    )
    def bcast(w_hbm, out_hbm, wv, buf, sem):
        wid = lax.axis_index("s") * info.num_cores + lax.axis_index("c")
        pltpu.sync_copy(w_hbm.at[0], wv.at[pl.ds(0, emb_dim)])
        pltpu.sync_copy(w_hbm.at[0], wv.at[pl.ds(emb_dim, emb_dim)])
        splats = [
            jnp.full((lanes,), wv[pl.ds(r, lanes)][0], jnp.float32)
            for r in range(SC_ROWS)
        ]

        def fill(j, _):
            for r in range(SC_ROWS):
                buf[r, pl.ds(j * lanes, lanes)] = splats[r]
            return 0

        lax.fori_loop(0, CHUNK_COLS // lanes, fill, 0)

        n_mine = (n_chunks - 1 - wid) // num_workers + 1

        def start(i, _):
            base = (wid + i * num_workers) * CHUNK_COLS
            pltpu.make_async_copy(
                buf, out_hbm.at[:, pl.ds(base, CHUNK_COLS)], sem
            ).start()
            return 0

        def drain(i, _):
            pltpu.make_async_copy(
                buf, out_hbm.at[:, pl.ds(wid * CHUNK_COLS, CHUNK_COLS)], sem
            ).wait()
            return 0

        lax.fori_loop(0, n_mine, start, 0)
        lax.fori_loop(0, n_mine, drain, 0)

    return bcast


def _tc_body(wt_ref, out_ref):
    out_ref[...] = jnp.broadcast_to(
        wt_ref[SC_ROWS:EMB_DIM, :], (EMB_DIM - SC_ROWS, TC_BLOCK_COLS)
    )


@functools.cache
def _build_tc_half(n_edges: int, emb_dim: int):
    assert n_edges % TC_BLOCK_COLS == 0
    grid = (n_edges // TC_BLOCK_COLS,)
    return pl.pallas_call(
        _tc_body,
        grid=grid,
        in_specs=[pl.BlockSpec((emb_dim, 1), lambda i: (0, 0))],
        out_specs=pl.BlockSpec(
            (emb_dim - SC_ROWS, TC_BLOCK_COLS), lambda i: (0, i)
        ),
        out_shape=jax.ShapeDtypeStruct(
            (emb_dim - SC_ROWS, n_edges), jnp.float32
        ),
    )


def kernel(batch, W):
    n_edges = batch.shape[0]
    sc_half = _build_sc_half(n_edges, EMB_DIM)(W)
    tc_half = _build_tc_half(n_edges, EMB_DIM)(W.T)
    return jnp.concatenate([sc_half, tc_half], axis=0).T


# speedup vs baseline: 2.2839x; 2.2839x over previous
import functools

import jax
import jax.numpy as jnp
from jax import lax
from jax.experimental import pallas as pl
from jax.experimental.pallas import tpu as pltpu
from jax.experimental.pallas import tpu_sc as plsc

EMB_DIM = 16
CHUNK_COLS = 2560


@functools.cache
def _build_broadcast(n_edges: int, emb_dim: int):
    info = plsc.get_sparse_core_info()
    num_workers = info.num_cores * info.num_subcores
    lanes = info.num_lanes
    assert n_edges % CHUNK_COLS == 0
    n_chunks = n_edges // CHUNK_COLS

    mesh = plsc.VectorSubcoreMesh(core_axis_name="c", subcore_axis_name="s")

    @functools.partial(
        pl.kernel,
        mesh=mesh,
        out_type=jax.ShapeDtypeStruct((emb_dim, n_edges), jnp.float32),
        scratch_types=[
            pltpu.VMEM((emb_dim, lanes), jnp.float32),
            pltpu.VMEM((emb_dim, CHUNK_COLS), jnp.float32),
            pltpu.SemaphoreType.DMA,
        ],
    )
    def bcast(splat_hbm, out_hbm, sv, buf, sem):
        wid = lax.axis_index("s") * info.num_cores + lax.axis_index("c")
        pltpu.sync_copy(splat_hbm, sv)
        splats = [sv[c] for c in range(emb_dim)]

        def fill(j, _):
            for c in range(emb_dim):
                buf[c, pl.ds(j * lanes, lanes)] = splats[c]
            return 0

        lax.fori_loop(0, CHUNK_COLS // lanes, fill, 0)

        n_mine = (n_chunks - 1 - wid) // num_workers + 1

        def start(i, _):
            base = (wid + i * num_workers) * CHUNK_COLS
            pltpu.make_async_copy(
                buf, out_hbm.at[:, pl.ds(base, CHUNK_COLS)], sem
            ).start()
            return 0

        def drain(i, _):
            pltpu.make_async_copy(
                buf, out_hbm.at[:, pl.ds(wid * CHUNK_COLS, CHUNK_COLS)], sem
            ).wait()
            return 0

        lax.fori_loop(0, n_mine, start, 0)
        lax.fori_loop(0, n_mine, drain, 0)

    return bcast


def kernel(batch, W):
    n_edges = batch.shape[0]
    splat = jnp.broadcast_to(W.reshape(EMB_DIM, 1), (EMB_DIM, EMB_DIM))
    cols = _build_broadcast(n_edges, EMB_DIM)(splat)
    return cols.T


# speedup vs baseline: 2.3092x; 1.0111x over previous
import functools

import jax
import jax.numpy as jnp
from jax import lax
from jax.experimental import pallas as pl
from jax.experimental.pallas import tpu as pltpu
from jax.experimental.pallas import tpu_sc as plsc

EMB_DIM = 16
CHUNK_COLS = 1280


@functools.cache
def _build_broadcast(n_edges: int, emb_dim: int):
    info = plsc.get_sparse_core_info()
    num_workers = info.num_cores * info.num_subcores
    lanes = info.num_lanes
    assert n_edges % CHUNK_COLS == 0
    n_chunks = n_edges // CHUNK_COLS

    mesh = plsc.VectorSubcoreMesh(core_axis_name="c", subcore_axis_name="s")

    @functools.partial(
        pl.kernel,
        mesh=mesh,
        out_type=jax.ShapeDtypeStruct((emb_dim, n_edges), jnp.float32),
        scratch_types=[
            pltpu.VMEM((emb_dim, lanes), jnp.float32),
            pltpu.VMEM((emb_dim, CHUNK_COLS), jnp.float32),
            pltpu.SemaphoreType.DMA,
        ],
    )
    def bcast(splat_hbm, out_hbm, sv, buf, sem):
        wid = lax.axis_index("s") * info.num_cores + lax.axis_index("c")
        pltpu.sync_copy(splat_hbm, sv)
        splats = [sv[c] for c in range(emb_dim)]

        def fill(j, _):
            for c in range(emb_dim):
                buf[c, pl.ds(j * lanes, lanes)] = splats[c]
            return 0

        lax.fori_loop(0, CHUNK_COLS // lanes, fill, 0)

        n_mine = (n_chunks - 1 - wid) // num_workers + 1

        def start(i, _):
            base = (wid + i * num_workers) * CHUNK_COLS
            pltpu.make_async_copy(
                buf, out_hbm.at[:, pl.ds(base, CHUNK_COLS)], sem
            ).start()
            return 0

        def drain(i, _):
            pltpu.make_async_copy(
                buf, out_hbm.at[:, pl.ds(wid * CHUNK_COLS, CHUNK_COLS)], sem
            ).wait()
            return 0

        lax.fori_loop(0, n_mine, start, 0)
        lax.fori_loop(0, n_mine, drain, 0)

    return bcast


def kernel(batch, W):
    n_edges = batch.shape[0]
    splat = jnp.broadcast_to(W.reshape(EMB_DIM, 1), (EMB_DIM, EMB_DIM))
    cols = _build_broadcast(n_edges, EMB_DIM)(splat)
    return cols.T
